# Initial kernel scaffold; baseline (speedup 1.0000x reference)
import jax
import jax.numpy as jnp
from jax.experimental import pallas as pl


def kernel(u, v, usr_table, ent_table, rel_table, adj_ent, adj_rel, W, b):
    raise NotImplementedError("write your pallas kernel here")


# speedup vs baseline: 14.7547x; 14.7547x over previous
import functools

import jax
import jax.numpy as jnp
from jax import lax
from jax.experimental import pallas as pl
from jax.experimental.pallas import tpu as pltpu
from jax.experimental.pallas import tpu_sc as plsc

NW = 32
DIM = 16
NN = 16


def _sfull_body(usr_ref, rel_ref, out_ref):
    out_ref[...] = lax.dot_general(
        usr_ref[...], rel_ref[...], (((1,), (1,)), ((), ())),
        preferred_element_type=jnp.float32)


def _sfull(usr_table, rel_table):
    nu, d = usr_table.shape
    nr = rel_table.shape[0]
    tr = 4000
    assert nu % tr == 0
    return pl.pallas_call(
        _sfull_body,
        grid=(nu // tr,),
        in_specs=[
            pl.BlockSpec((tr, d), lambda i: (i, 0)),
            pl.BlockSpec((nr, d), lambda i: (0, 0)),
        ],
        out_specs=pl.BlockSpec((tr, nr), lambda i: (i, 0)),
        out_shape=jax.ShapeDtypeStruct((nu, nr), jnp.float32),
    )(usr_table, rel_table)


def _sc_gather(u, v, usr_table, ent_table, adj_ent, adj_rel, s_full):
    B = u.shape[0]
    nb = B // NW
    nc1 = nb * NN // 128
    ipc = 128 // NN
    nc2 = nb * NN * NN // 128
    n_stage = 8
    mesh = plsc.VectorSubcoreMesh(core_axis_name="c", subcore_axis_name="s")

    @functools.partial(
        pl.kernel,
        out_type=[
            jax.ShapeDtypeStruct((NW, nb, DIM), jnp.float32),
            jax.ShapeDtypeStruct((NW, nb, DIM), jnp.float32),
            jax.ShapeDtypeStruct((NW, nb * NN, DIM), jnp.float32),
            jax.ShapeDtypeStruct((NW, nb * NN * NN, DIM), jnp.float32),
            jax.ShapeDtypeStruct((NW, nb, NN), jnp.float32),
            jax.ShapeDtypeStruct((NW, nb, NN * NN), jnp.float32),
        ],
        mesh=mesh,
        compiler_params=pltpu.CompilerParams(
            needs_layout_passes=False, use_tc_tiling_on_sc=False),
        scratch_types=[
            pltpu.VMEM((nb,), jnp.int32),
            pltpu.VMEM((nb,), jnp.int32),
            pltpu.VMEM((nb, 32), jnp.float32),
            pltpu.VMEM((nb, DIM), jnp.float32),
            pltpu.VMEM((nb, DIM), jnp.float32),
            pltpu.VMEM((nb, NN), jnp.int32),
            pltpu.VMEM((nb, NN), jnp.int32),
            pltpu.VMEM((nc1, 128), jnp.int32),
            pltpu.VMEM((128, DIM), jnp.float32),
            pltpu.VMEM((128, NN), jnp.int32),
            pltpu.VMEM((128, NN), jnp.int32),
            pltpu.VMEM((nc2, 128), jnp.int32),
            pltpu.VMEM((ipc, NN * NN), jnp.float32),
            pltpu.VMEM((nb, NN), jnp.float32),
            pltpu.VMEM((n_stage * 128, DIM), jnp.float32),
            pltpu.SemaphoreType.DMA,
        ],
    )
    def body(u_hbm, v_hbm, usr_hbm, ent_hbm, adj_ent_hbm, adj_rel_hbm, sf_hbm,
             ue_out, ev0_out, ev1_out, ev2_out, s0_out, s1_out,
             u_v, v_v, srow_v, ue_v, ev0_v, r0_v, e1_v, e1f_v,
             ev1c_v, r1c_v, e2c_v, e2f_v, s1c_v, s0_v, stage_v, sem):
        wid = lax.axis_index("s") * 2 + lax.axis_index("c")
        base = wid * nb

        pltpu.sync_copy(u_hbm.at[pl.ds(base, nb)], u_v)
        pltpu.sync_copy(v_hbm.at[pl.ds(base, nb)], v_v)

        d1 = pltpu.async_copy(sf_hbm.at[u_v], srow_v, sem)
        d2 = pltpu.async_copy(usr_hbm.at[u_v], ue_v, sem)
        d3 = pltpu.async_copy(ent_hbm.at[v_v], ev0_v, sem)
        d4 = pltpu.async_copy(adj_rel_hbm.at[v_v], r0_v, sem)
        d5 = pltpu.async_copy(adj_ent_hbm.at[v_v], e1_v, sem)
        d1.wait(); d2.wait(); d3.wait(); d4.wait(); d5.wait()
        pltpu.sync_copy(ue_v, ue_out.at[wid])
        pltpu.sync_copy(ev0_v, ev0_out.at[wid])

        def s0_body(i, _):
            rvec = r0_v[i]
            isplat = jnp.full((NN,), i, dtype=jnp.int32)
            s0_v[i] = plsc.load_gather(srow_v, [isplat, rvec])
            return _
        lax.fori_loop(0, nb, s0_body, 0)
        pltpu.sync_copy(s0_v, s0_out.at[wid])

        def e1f_body(j, _):
            for k in range(128 // NN):
                e1f_v[j, pl.ds(k * NN, NN)] = e1_v[j * (128 // NN) + k]
            return _
        lax.fori_loop(0, nc1, e1f_body, 0)

        def lvl1_body(c, _):
            idx = e1f_v.at[c]
            g1 = pltpu.async_copy(ent_hbm.at[idx], ev1c_v, sem)
            g2 = pltpu.async_copy(adj_rel_hbm.at[idx], r1c_v, sem)
            g3 = pltpu.async_copy(adj_ent_hbm.at[idx], e2c_v, sem)
            g1.wait(); g2.wait(); g3.wait()
            pltpu.sync_copy(ev1c_v, ev1_out.at[wid, pl.ds(c * 128, 128)])
            for k in range(ipc):
                i = c * ipc + k
                isplat = jnp.full((NN,), i, dtype=jnp.int32)
                for g in range(NN):
                    rvec = r1c_v[k * NN + g]
                    s1c_v[k, pl.ds(g * NN, NN)] = plsc.load_gather(
                        srow_v, [isplat, rvec])
            pltpu.sync_copy(s1c_v, s1_out.at[wid, pl.ds(c * ipc, ipc)])
            for jj in range(128 * NN // 128):
                for k in range(128 // NN):
                    e2f_v[c * (128 * NN // 128) + jj, pl.ds(k * NN, NN)] = (
                        e2c_v[jj * (128 // NN) + k])
            return _
        lax.fori_loop(0, nc1, lvl1_body, 0)

        def lvl2_body(j, _):
            ds = []
            for k in range(n_stage):
                ds.append(pltpu.async_copy(
                    ent_hbm.at[e2f_v.at[j * n_stage + k]],
                    stage_v.at[pl.ds(k * 128, 128)], sem))
            for dd in ds:
                dd.wait()
            pltpu.sync_copy(
                stage_v,
                ev2_out.at[wid, pl.ds(j * (n_stage * 128), n_stage * 128)])
            return _
        lax.fori_loop(0, nc2 // n_stage, lvl2_body, 0)

    return body(u, v, usr_table, ent_table, adj_ent, adj_rel, s_full)


def _iota_e():
    return (lax.broadcasted_iota(jnp.int32, (NN, NN * DIM), 1) // DIM ==
            lax.broadcasted_iota(jnp.int32, (NN, NN * DIM), 0)
            ).astype(jnp.float32)


def _iota_s():
    return (lax.broadcasted_iota(jnp.int32, (NN * DIM, DIM), 0) % DIM ==
            lax.broadcasted_iota(jnp.int32, (NN * DIM, DIM), 1)
            ).astype(jnp.float32)


def _attn(s, selfv, neigh, W, bb):
    m = jnp.max(s, axis=-1, keepdims=True)
    e = jnp.exp(s - m)
    w = e / jnp.sum(e, axis=-1, keepdims=True)
    wl = jnp.dot(w, _iota_e(), preferred_element_type=jnp.float32)
    t = wl * neigh
    agg = jnp.dot(t, _iota_s(), preferred_element_type=jnp.float32)
    return jnp.dot(selfv + agg, W, preferred_element_type=jnp.float32) + bb


def _dense_body(s1_ref, ev1f_ref, ev2_ref, s0_ref, ev0_ref, ev1v_ref,
                w_ref, b_ref, h1_ref, h0_ref):
    W = w_ref[...]
    bb = b_ref[...]
    h1_ref[...] = jnp.maximum(
        _attn(s1_ref[...], ev1f_ref[...], ev2_ref[...], W, bb), 0.0)
    h0_ref[...] = jnp.maximum(
        _attn(s0_ref[...], ev0_ref[...], ev1v_ref[...], W, bb), 0.0)


def _dense(s1, ev1f, ev2v, s0, ev0, ev1v, W, bb):
    B = s0.shape[0]
    G = B * NN
    nt = 16
    tb = B // nt
    return pl.pallas_call(
        _dense_body,
        grid=(nt,),
        in_specs=[
            pl.BlockSpec((tb * NN, NN), lambda i: (i, 0)),
            pl.BlockSpec((tb * NN, DIM), lambda i: (i, 0)),
            pl.BlockSpec((tb * NN, NN * DIM), lambda i: (i, 0)),
            pl.BlockSpec((tb, NN), lambda i: (i, 0)),
            pl.BlockSpec((tb, DIM), lambda i: (i, 0)),
            pl.BlockSpec((tb, NN * DIM), lambda i: (i, 0)),
            pl.BlockSpec((DIM, DIM), lambda i: (0, 0)),
            pl.BlockSpec((1, DIM), lambda i: (0, 0)),
        ],
        out_specs=[
            pl.BlockSpec((tb * NN, DIM), lambda i: (i, 0)),
            pl.BlockSpec((tb, DIM), lambda i: (i, 0)),
        ],
        out_shape=[
            jax.ShapeDtypeStruct((G, DIM), jnp.float32),
            jax.ShapeDtypeStruct((B, DIM), jnp.float32),
        ],
    )(s1, ev1f, ev2v, s0, ev0, ev1v, W, bb)


def _final_body(s0_ref, h0_ref, h1v_ref, ue_ref, w_ref, b_ref, out_ref):
    item = jnp.tanh(_attn(s0_ref[...], h0_ref[...], h1v_ref[...],
                          w_ref[...], b_ref[...]))
    logit = jnp.sum(ue_ref[...] * item, axis=-1, keepdims=True)
    out_ref[...] = jax.nn.sigmoid(logit)


def _final(s0, h0, h1v, ue, W, bb):
    B = s0.shape[0]
    nt = 4
    tb = B // nt
    return pl.pallas_call(
        _final_body,
        grid=(nt,),
        in_specs=[
            pl.BlockSpec((tb, NN), lambda i: (i, 0)),
            pl.BlockSpec((tb, DIM), lambda i: (i, 0)),
            pl.BlockSpec((tb, NN * DIM), lambda i: (i, 0)),
            pl.BlockSpec((tb, DIM), lambda i: (i, 0)),
            pl.BlockSpec((DIM, DIM), lambda i: (0, 0)),
            pl.BlockSpec((1, DIM), lambda i: (0, 0)),
        ],
        out_specs=pl.BlockSpec((tb, 1), lambda i: (i, 0)),
        out_shape=jax.ShapeDtypeStruct((B, 1), jnp.float32),
    )(s0, h0, h1v, ue, W, bb)


def kernel(u, v, usr_table, ent_table, rel_table, adj_ent, adj_rel, W, b):
    B = u.shape[0]
    s_full = _sfull(usr_table, rel_table)
    ue, ev0, ev1, ev2, s0, s1 = _sc_gather(
        u, v, usr_table, ent_table, adj_ent, adj_rel, s_full)
    ue = ue.reshape(B, DIM)
    ev0 = ev0.reshape(B, DIM)
    ev1f = ev1.reshape(B * NN, DIM)
    ev1v = ev1.reshape(B, NN * DIM)
    ev2v = ev2.reshape(B * NN, NN * DIM)
    s0 = s0.reshape(B, NN)
    s1 = s1.reshape(B * NN, NN)
    bb = b.reshape(1, DIM)
    h1, h0 = _dense(s1, ev1f, ev2v, s0, ev0, ev1v, W, bb)
    h1v = h1.reshape(B, NN * DIM)
    out = _final(s0, h0, h1v, ue, W, bb)
    return out.reshape(B)


# speedup vs baseline: 20.4401x; 1.3853x over previous
import functools

import jax
import jax.numpy as jnp
from jax import lax
from jax.experimental import pallas as pl
from jax.experimental.pallas import tpu as pltpu
from jax.experimental.pallas import tpu_sc as plsc

NW = 32
DIM = 16
NN = 16
NR = 32


def _sc_phase(u, v, usr_table, ent_table, rel_table, adj_ent, adj_rel):
    B = u.shape[0]
    nb = B // NW
    nc1 = nb * NN // 128
    ipc = 128 // NN
    mesh = plsc.VectorSubcoreMesh(core_axis_name="c", subcore_axis_name="s")

    @functools.partial(
        pl.kernel,
        out_type=[
            jax.ShapeDtypeStruct((NW, nb, DIM), jnp.float32),
            jax.ShapeDtypeStruct((NW, nb, NN), jnp.float32),
            jax.ShapeDtypeStruct((NW, nb, DIM), jnp.float32),
            jax.ShapeDtypeStruct((NW, nb, NN * DIM), jnp.float32),
        ],
        mesh=mesh,
        compiler_params=pltpu.CompilerParams(
            needs_layout_passes=False, use_tc_tiling_on_sc=False),
        scratch_types=[
            pltpu.VMEM((nb,), jnp.int32),
            pltpu.VMEM((nb,), jnp.int32),
            pltpu.VMEM((NR, DIM), jnp.float32),
            pltpu.VMEM((DIM, NR), jnp.float32),
            pltpu.VMEM((nb, NR), jnp.float32),
            pltpu.VMEM((nb, DIM), jnp.float32),
            pltpu.VMEM((nb, DIM), jnp.float32),
            pltpu.VMEM((nb, NN), jnp.int32),
            pltpu.VMEM((nb, NN), jnp.int32),
            pltpu.VMEM((nc1, 128), jnp.int32),
            pltpu.VMEM((nb, NN), jnp.float32),
            pltpu.VMEM((nb, DIM), jnp.float32),
            pltpu.VMEM((128, DIM), jnp.float32),
            pltpu.VMEM((128, NN), jnp.int32),
            pltpu.VMEM((128, NN), jnp.int32),
            pltpu.VMEM((NN, 128), jnp.int32),
            pltpu.VMEM((ipc, NN * DIM), jnp.float32),
            pltpu.VMEM((2, 128, DIM), jnp.float32),
            pltpu.SemaphoreType.DMA,
        ],
    )
    def body(u_hbm, v_hbm, usr_hbm, ent_hbm, rel_hbm, adj_ent_hbm, adj_rel_hbm,
             ue_out, s0_out, sa0_out, sa1_out,
             u_v, v_v, rel_v, relT_v, srow_v, ue_v, ev0_v, r0_v, e1_v, e1f_v,
             s0_v, sa0_v, ev1c_v, r1c_v, e2c_v, e2f_c, sac_v, stage_v, sem):
        wid = lax.axis_index("s") * 2 + lax.axis_index("c")
        base = wid * nb

        pltpu.sync_copy(u_hbm.at[pl.ds(base, nb)], u_v)
        pltpu.sync_copy(v_hbm.at[pl.ds(base, nb)], v_v)
        pltpu.sync_copy(rel_hbm, rel_v)

        d2 = pltpu.async_copy(usr_hbm.at[u_v], ue_v, sem)
        d3 = pltpu.async_copy(ent_hbm.at[v_v], ev0_v, sem)
        d4 = pltpu.async_copy(adj_rel_hbm.at[v_v], r0_v, sem)
        d5 = pltpu.async_copy(adj_ent_hbm.at[v_v], e1_v, sem)

        dim_iota = lax.iota(jnp.int32, DIM)
        for r in range(NR):
            plsc.store_scatter(
                relT_v, [dim_iota, jnp.full((DIM,), r, dtype=jnp.int32)],
                rel_v[r])

        d2.wait(); d3.wait(); d4.wait(); d5.wait()
        pltpu.sync_copy(ue_v, ue_out.at[wid])

        def srow_body(i, _):
            uev = ue_v[i]
            a0 = uev[0] * relT_v[0, pl.ds(0, NN)]
            a1 = uev[0] * relT_v[0, pl.ds(NN, NN)]
            for dd in range(1, DIM):
                a0 = a0 + uev[dd] * relT_v[dd, pl.ds(0, NN)]
                a1 = a1 + uev[dd] * relT_v[dd, pl.ds(NN, NN)]
            srow_v[i, pl.ds(0, NN)] = a0
            srow_v[i, pl.ds(NN, NN)] = a1
            isplat = jnp.full((NN,), i, dtype=jnp.int32)
            s0_v[i] = plsc.load_gather(srow_v, [isplat, r0_v[i]])
            return _
        lax.fori_loop(0, nb, srow_body, 0)
        pltpu.sync_copy(s0_v, s0_out.at[wid])

        def e1f_body(j, _):
            for k in range(128 // NN):
                e1f_v[j, pl.ds(k * NN, NN)] = e1_v[j * (128 // NN) + k]
            return _
        lax.fori_loop(0, nc1, e1f_body, 0)

        def softmax16(svec):
            m = jnp.max(svec)
            e = jnp.exp(svec - m)
            return e / jnp.sum(e)

        def lvl1_body(c, _):
            idx = e1f_v.at[c]
            g1 = pltpu.async_copy(ent_hbm.at[idx], ev1c_v, sem)
            g2 = pltpu.async_copy(adj_rel_hbm.at[idx], r1c_v, sem)
            g3 = pltpu.async_copy(adj_ent_hbm.at[idx], e2c_v, sem)
            g1.wait(); g2.wait(); g3.wait()

            def agg0_body(k, _2):
                i = c * ipc + k
                wv = softmax16(s0_v[i])
                acc = ev0_v[i] + wv[0] * ev1c_v[k * NN]
                for n in range(1, NN):
                    acc = acc + wv[n] * ev1c_v[k * NN + n]
                sa0_v[i] = acc
                return _2
            lax.fori_loop(0, ipc, agg0_body, 0)

            def e2f_body(jj, _2):
                for k in range(128 // NN):
                    e2f_c[jj, pl.ds(k * NN, NN)] = e2c_v[jj * (128 // NN) + k]
                return _2
            lax.fori_loop(0, NN, e2f_body, 0)

            pltpu.async_copy(ent_hbm.at[e2f_c.at[0]], stage_v.at[0], sem)

            def lvl2_body(j, _2):
                jm = j % 2
                pltpu.make_async_copy(
                    ent_hbm.at[e2f_c.at[j]], stage_v.at[jm], sem).wait()

                @pl.when(j < NN - 1)
                def _fire():
                    pltpu.async_copy(
                        ent_hbm.at[e2f_c.at[j + 1]],
                        stage_v.at[(j + 1) % 2], sem)

                for k2 in range(ipc):
                    gg = j * ipc + k2
                    isplat = jnp.full(
                        (NN,), c * ipc + gg // NN, dtype=jnp.int32)
                    svec = plsc.load_gather(srow_v, [isplat, r1c_v[gg]])
                    wv = softmax16(svec)
                    acc = ev1c_v[gg] + wv[0] * stage_v[jm, k2 * NN]
                    for n in range(1, NN):
                        acc = acc + wv[n] * stage_v[jm, k2 * NN + n]
                    sac_v[gg // NN, pl.ds((gg % NN) * DIM, DIM)] = acc
                return _2
            lax.fori_loop(0, NN, lvl2_body, 0)
            pltpu.sync_copy(sac_v, sa1_out.at[wid, pl.ds(c * ipc, ipc)])
            return _
        lax.fori_loop(0, nc1, lvl1_body, 0)
        pltpu.sync_copy(sa0_v, sa0_out.at[wid])

    return body(u, v, usr_table, ent_table, rel_table, adj_ent, adj_rel)


def _head_body(ue_ref, s0_ref, sa0_ref, sa1_ref, w_ref, b_ref, out_ref):
    W = w_ref[...]
    bvec = b_ref[...]
    f32 = jnp.float32
    E = (lax.broadcasted_iota(jnp.int32, (NN, NN * DIM), 1) // DIM ==
         lax.broadcasted_iota(jnp.int32, (NN, NN * DIM), 0)).astype(f32)
    S = (lax.broadcasted_iota(jnp.int32, (NN * DIM, DIM), 0) % DIM ==
         lax.broadcasted_iota(jnp.int32, (NN * DIM, DIM), 1)).astype(f32)
    P = (lax.broadcasted_iota(jnp.int32, (DIM, NN * DIM), 1) % DIM ==
         lax.broadcasted_iota(jnp.int32, (DIM, NN * DIM), 0)).astype(f32)
    blockmask = (
        lax.broadcasted_iota(jnp.int32, (NN * DIM, NN * DIM), 0) // DIM ==
        lax.broadcasted_iota(jnp.int32, (NN * DIM, NN * DIM), 1) // DIM
    ).astype(f32)
    BD = jnp.dot(S, jnp.dot(W, P, preferred_element_type=f32),
                 preferred_element_type=f32) * blockmask
    btile = jnp.dot(bvec, P, preferred_element_type=f32)

    sa1 = sa1_ref[...]
    h1v = jnp.maximum(
        jnp.dot(sa1, BD, preferred_element_type=f32) + btile, 0.0)
    h0 = jnp.maximum(
        jnp.dot(sa0_ref[...], W, preferred_element_type=f32) + bvec, 0.0)

    s0 = s0_ref[...]
    m = jnp.max(s0, axis=-1, keepdims=True)
    e = jnp.exp(s0 - m)
    w0 = e / jnp.sum(e, axis=-1, keepdims=True)
    t2 = jnp.dot(w0, E, preferred_element_type=f32) * h1v
    agg = jnp.dot(t2, S, preferred_element_type=f32)
    item = jnp.tanh(jnp.dot(h0 + agg, W, preferred_element_type=f32) + bvec)
    logit = jnp.sum(ue_ref[...] * item, axis=-1, keepdims=True)
    out_ref[...] = jax.nn.sigmoid(logit)


def _head(ue, s0, sa0, sa1v, W, bb):
    B = s0.shape[0]
    nt = 4
    tb = B // nt
    return pl.pallas_call(
        _head_body,
        grid=(nt,),
        in_specs=[
            pl.BlockSpec((tb, DIM), lambda i: (i, 0)),
            pl.BlockSpec((tb, NN), lambda i: (i, 0)),
            pl.BlockSpec((tb, DIM), lambda i: (i, 0)),
            pl.BlockSpec((tb, NN * DIM), lambda i: (i, 0)),
            pl.BlockSpec((DIM, DIM), lambda i: (0, 0)),
            pl.BlockSpec((1, DIM), lambda i: (0, 0)),
        ],
        out_specs=pl.BlockSpec((tb, 1), lambda i: (i, 0)),
        out_shape=jax.ShapeDtypeStruct((B, 1), jnp.float32),
    )(ue, s0, sa0, sa1v, W, bb)


def kernel(u, v, usr_table, ent_table, rel_table, adj_ent, adj_rel, W, b):
    B = u.shape[0]
    ue, s0, sa0, sa1 = _sc_phase(
        u, v, usr_table, ent_table, rel_table, adj_ent, adj_rel)
    ue = ue.reshape(B, DIM)
    s0 = s0.reshape(B, NN)
    sa0 = sa0.reshape(B, DIM)
    sa1v = sa1.reshape(B, NN * DIM)
    bb = b.reshape(1, DIM)
    out = _head(ue, s0, sa0, sa1v, W, bb)
    return out.reshape(B)


# speedup vs baseline: 30.9878x; 1.5160x over previous
import functools

import jax
import jax.numpy as jnp
from jax import lax
from jax.experimental import pallas as pl
from jax.experimental.pallas import tpu as pltpu
from jax.experimental.pallas import tpu_sc as plsc

NW = 32
DIM = 16
NN = 16
NR = 32

_SC_PARAMS = pltpu.CompilerParams(
    needs_layout_passes=False, use_tc_tiling_on_sc=False)


def _sc_expand(u, v, usr_table, rel_table, adj_ent, adj_rel):
    B = u.shape[0]
    nb = B // NW
    nc1 = nb * NN // 128
    ipc = 128 // NN
    mesh = plsc.VectorSubcoreMesh(core_axis_name="c", subcore_axis_name="s")

    @functools.partial(
        pl.kernel,
        out_type=[
            jax.ShapeDtypeStruct((NW, nb, DIM), jnp.float32),
            jax.ShapeDtypeStruct((NW, nb, NN), jnp.float32),
            jax.ShapeDtypeStruct((NW, nb, NN * NN), jnp.float32),
            jax.ShapeDtypeStruct((NW, nc1, 128), jnp.int32),
            jax.ShapeDtypeStruct((NW, nc1 * NN, 128), jnp.int32),
        ],
        mesh=mesh,
        compiler_params=_SC_PARAMS,
        scratch_types=[
            pltpu.VMEM((nb,), jnp.int32),
            pltpu.VMEM((nb,), jnp.int32),
            pltpu.VMEM((NR, DIM), jnp.float32),
            pltpu.VMEM((DIM, NR), jnp.float32),
            pltpu.VMEM((nb, NR), jnp.float32),
            pltpu.VMEM((nb, DIM), jnp.float32),
            pltpu.VMEM((nb, NN), jnp.int32),
            pltpu.VMEM((nb, NN), jnp.int32),
            pltpu.VMEM((nc1, 128), jnp.int32),
            pltpu.VMEM((nb, NN), jnp.float32),
            pltpu.VMEM((2, 128, NN), jnp.int32),
            pltpu.VMEM((2, 128, NN), jnp.int32),
            pltpu.VMEM((NN, 128), jnp.int32),
            pltpu.VMEM((ipc, NN * NN), jnp.float32),
            pltpu.SemaphoreType.DMA,
        ],
    )
    def body(u_hbm, v_hbm, usr_hbm, rel_hbm, adj_ent_hbm, adj_rel_hbm,
             ue_out, s0_out, s1_out, e1f_out, e2f_out,
             u_v, v_v, rel_v, relT_v, srow_v, ue_v, r0_v, e1_v, e1f_v, s0_v,
             r1c_v, e2c_v, e2f_c, s1c_v, sem):
        wid = lax.axis_index("s") * 2 + lax.axis_index("c")
        base = wid * nb

        pltpu.sync_copy(u_hbm.at[pl.ds(base, nb)], u_v)
        pltpu.sync_copy(v_hbm.at[pl.ds(base, nb)], v_v)
        pltpu.sync_copy(rel_hbm, rel_v)

        d2 = pltpu.async_copy(usr_hbm.at[u_v], ue_v, sem)
        d4 = pltpu.async_copy(adj_rel_hbm.at[v_v], r0_v, sem)
        d5 = pltpu.async_copy(adj_ent_hbm.at[v_v], e1_v, sem)

        dim_iota = lax.iota(jnp.int32, DIM)
        for r in range(NR):
            plsc.store_scatter(
                relT_v, [dim_iota, jnp.full((DIM,), r, dtype=jnp.int32)],
                rel_v[r])

        d2.wait(); d4.wait(); d5.wait()
        pltpu.sync_copy(ue_v, ue_out.at[wid])

        def _tree_dot(uev, half):
            parts = []
            for q in range(4):
                t = uev[4 * q] * relT_v[4 * q, pl.ds(half * NN, NN)]
                for dd in range(4 * q + 1, 4 * q + 4):
                    t = t + uev[dd] * relT_v[dd, pl.ds(half * NN, NN)]
                parts.append(t)
            return (parts[0] + parts[1]) + (parts[2] + parts[3])

        @plsc.parallel_loop(0, nb, unroll=4)
        def srow_body(i):
            uev = ue_v[i]
            srow_v[i, pl.ds(0, NN)] = _tree_dot(uev, 0)
            srow_v[i, pl.ds(NN, NN)] = _tree_dot(uev, 1)

        @plsc.parallel_loop(0, nb, unroll=4)
        def s0_body(i):
            isplat = jnp.full((NN,), i, dtype=jnp.int32)
            s0_v[i] = plsc.load_gather(srow_v, [isplat, r0_v[i]])
        pltpu.sync_copy(s0_v, s0_out.at[wid])

        def e1f_body(j, _):
            for k in range(128 // NN):
                e1f_v[j, pl.ds(k * NN, NN)] = e1_v[j * (128 // NN) + k]
            return _
        lax.fori_loop(0, nc1, e1f_body, 0)
        pltpu.sync_copy(e1f_v, e1f_out.at[wid])

        pltpu.async_copy(adj_rel_hbm.at[e1f_v.at[0]], r1c_v.at[0], sem)
        pltpu.async_copy(adj_ent_hbm.at[e1f_v.at[0]], e2c_v.at[0], sem)

        def lvl1_body(c, _):
            cm = c % 2
            pltpu.make_async_copy(
                adj_rel_hbm.at[e1f_v.at[c]], r1c_v.at[cm], sem).wait()
            pltpu.make_async_copy(
                adj_ent_hbm.at[e1f_v.at[c]], e2c_v.at[cm], sem).wait()

            @pl.when(c < nc1 - 1)
            def _fire():
                cn = (c + 1) % 2
                pltpu.async_copy(
                    adj_rel_hbm.at[e1f_v.at[c + 1]], r1c_v.at[cn], sem)
                pltpu.async_copy(
                    adj_ent_hbm.at[e1f_v.at[c + 1]], e2c_v.at[cn], sem)

            @plsc.parallel_loop(0, 128, unroll=8)
            def s1_body(g):
                i = c * ipc + g // NN
                isplat = jnp.full((NN,), i, dtype=jnp.int32)
                vals = plsc.load_gather(srow_v, [isplat, r1c_v[cm, g]])
                s1c_v[(g // NN) % ipc, pl.ds((g % NN) * NN, NN)] = vals
            pltpu.sync_copy(s1c_v, s1_out.at[wid, pl.ds(c * ipc, ipc)])

            @plsc.parallel_loop(0, NN, unroll=4)
            def e2f_body(jj):
                for k in range(128 // NN):
                    e2f_c[jj, pl.ds(k * NN, NN)] = (
                        e2c_v[cm, jj * (128 // NN) + k])
            pltpu.sync_copy(e2f_c, e2f_out.at[wid, pl.ds(c * NN, NN)])
            return _
        lax.fori_loop(0, nc1, lvl1_body, 0)

    return body(u, v, usr_table, rel_table, adj_ent, adj_rel)


def _sc_aggregate(v, ent_table, s0, s1, e1f, e2f):
    B = v.shape[0]
    nb = B // NW
    nc1 = nb * NN // 128
    ipc = 128 // NN
    mesh = plsc.VectorSubcoreMesh(core_axis_name="c", subcore_axis_name="s")

    @functools.partial(
        pl.kernel,
        out_type=[
            jax.ShapeDtypeStruct((NW, nb, DIM), jnp.float32),
            jax.ShapeDtypeStruct((NW, nb, NN * DIM), jnp.float32),
        ],
        mesh=mesh,
        compiler_params=_SC_PARAMS,
        scratch_types=[
            pltpu.VMEM((nb,), jnp.int32),
            pltpu.VMEM((nb, NN), jnp.float32),
            pltpu.VMEM((nb, DIM), jnp.float32),
            pltpu.VMEM((nb, DIM), jnp.float32),
            pltpu.VMEM((nc1, 128), jnp.int32),
            pltpu.VMEM((NN, 128), jnp.int32),
            pltpu.VMEM((ipc, NN * NN), jnp.float32),
            pltpu.VMEM((128, DIM), jnp.float32),
            pltpu.VMEM((ipc, NN * DIM), jnp.float32),
            pltpu.VMEM((NN, 128, DIM), jnp.float32),
            pltpu.SemaphoreType.DMA,
            pltpu.SemaphoreType.DMA,
        ],
    )
    def body(v_hbm, ent_hbm, s0_hbm, s1_hbm, e1f_hbm, e2f_hbm,
             sa0_out, sa1_out,
             v_v, s0_v, ev0_v, sa0_v, e1f_v, e2f_c, s1c_v, ev1c_v, sac_v,
             stage_v, sem, sem2):
        wid = lax.axis_index("s") * 2 + lax.axis_index("c")
        base = wid * nb

        pltpu.sync_copy(v_hbm.at[pl.ds(base, nb)], v_v)
        pltpu.sync_copy(s0_hbm.at[wid], s0_v)
        pltpu.sync_copy(e1f_hbm.at[wid], e1f_v)
        d3 = pltpu.async_copy(ent_hbm.at[v_v], ev0_v, sem)
        d3.wait()

        def expz(svec):
            e = jnp.exp(svec - jnp.max(svec))
            return e, jnp.sum(e)

        def wsum(wv, rows):
            parts = []
            for q in range(4):
                t = wv[4 * q] * rows[4 * q]
                for n in range(4 * q + 1, 4 * q + 4):
                    t = t + wv[n] * rows[n]
                parts.append(t)
            return (parts[0] + parts[1]) + (parts[2] + parts[3])

        def lvl1_body(c, _):
            idx = e1f_v.at[c]
            g1 = pltpu.async_copy(ent_hbm.at[idx], ev1c_v, sem)
            pltpu.sync_copy(s1_hbm.at[wid, pl.ds(c * ipc, ipc)], s1c_v)
            pltpu.sync_copy(e2f_hbm.at[wid, pl.ds(c * NN, NN)], e2f_c)
            for jf in range(NN):
                pltpu.async_copy(
                    ent_hbm.at[e2f_c.at[jf]], stage_v.at[jf], sem2)
            g1.wait()

            @plsc.parallel_loop(0, ipc, unroll=ipc)
            def agg0_body(k):
                i = c * ipc + k
                ev, z = expz(s0_v[i])
                rows = [ev1c_v[k * NN + n] for n in range(NN)]
                sa0_v[i] = ev0_v[i] + wsum(ev, rows) / z

            def lvl2_body(j, _2):
                pltpu.make_async_copy(
                    ent_hbm.at[e2f_c.at[j]], stage_v.at[j], sem2).wait()

                @plsc.parallel_loop(0, ipc, unroll=ipc)
                def grp_body(k2):
                    gg = j * ipc + k2
                    svec = s1c_v[gg // NN, pl.ds((gg % NN) * NN, NN)]
                    ev, z = expz(svec)
                    rows = [stage_v[j, k2 * NN + n] for n in range(NN)]
                    acc = ev1c_v[gg] + wsum(ev, rows) / z
                    sac_v[gg // NN, pl.ds((gg % NN) * DIM, DIM)] = acc
                return _2
            lax.fori_loop(0, NN, lvl2_body, 0)
            pltpu.sync_copy(sac_v, sa1_out.at[wid, pl.ds(c * ipc, ipc)])
            return _
        lax.fori_loop(0, nc1, lvl1_body, 0)
        pltpu.sync_copy(sa0_v, sa0_out.at[wid])

    return body(v, ent_table, s0, s1, e1f, e2f)


def _head_body(ue_ref, s0_ref, sa0_ref, sa1_ref, w_ref, b_ref, out_ref):
    W = w_ref[...]
    bvec = b_ref[...]
    f32 = jnp.float32
    E = (lax.broadcasted_iota(jnp.int32, (NN, NN * DIM), 1) // DIM ==
         lax.broadcasted_iota(jnp.int32, (NN, NN * DIM), 0)).astype(f32)
    S = (lax.broadcasted_iota(jnp.int32, (NN * DIM, DIM), 0) % DIM ==
         lax.broadcasted_iota(jnp.int32, (NN * DIM, DIM), 1)).astype(f32)
    P = (lax.broadcasted_iota(jnp.int32, (DIM, NN * DIM), 1) % DIM ==
         lax.broadcasted_iota(jnp.int32, (DIM, NN * DIM), 0)).astype(f32)
    blockmask = (
        lax.broadcasted_iota(jnp.int32, (NN * DIM, NN * DIM), 0) // DIM ==
        lax.broadcasted_iota(jnp.int32, (NN * DIM, NN * DIM), 1) // DIM
    ).astype(f32)
    BD = jnp.dot(S, jnp.dot(W, P, preferred_element_type=f32),
                 preferred_element_type=f32) * blockmask
    btile = jnp.dot(bvec, P, preferred_element_type=f32)

    sa1 = sa1_ref[...]
    h1v = jnp.maximum(
        jnp.dot(sa1, BD, preferred_element_type=f32) + btile, 0.0)
    h0 = jnp.maximum(
        jnp.dot(sa0_ref[...], W, preferred_element_type=f32) + bvec, 0.0)

    s0 = s0_ref[...]
    m = jnp.max(s0, axis=-1, keepdims=True)
    e = jnp.exp(s0 - m)
    w0 = e / jnp.sum(e, axis=-1, keepdims=True)
    t2 = jnp.dot(w0, E, preferred_element_type=f32) * h1v
    agg = jnp.dot(t2, S, preferred_element_type=f32)
    item = jnp.tanh(jnp.dot(h0 + agg, W, preferred_element_type=f32) + bvec)
    logit = jnp.sum(ue_ref[...] * item, axis=-1, keepdims=True)
    out_ref[...] = jax.nn.sigmoid(logit)


def _head(ue, s0, sa0, sa1v, W, bb):
    B = s0.shape[0]
    nt = 4
    tb = B // nt
    return pl.pallas_call(
        _head_body,
        grid=(nt,),
        in_specs=[
            pl.BlockSpec((tb, DIM), lambda i: (i, 0)),
            pl.BlockSpec((tb, NN), lambda i: (i, 0)),
            pl.BlockSpec((tb, DIM), lambda i: (i, 0)),
            pl.BlockSpec((tb, NN * DIM), lambda i: (i, 0)),
            pl.BlockSpec((DIM, DIM), lambda i: (0, 0)),
            pl.BlockSpec((1, DIM), lambda i: (0, 0)),
        ],
        out_specs=pl.BlockSpec((tb, 1), lambda i: (i, 0)),
        out_shape=jax.ShapeDtypeStruct((B, 1), jnp.float32),
    )(ue, s0, sa0, sa1v, W, bb)


def kernel(u, v, usr_table, ent_table, rel_table, adj_ent, adj_rel, W, b):
    B = u.shape[0]
    ue, s0, s1, e1f, e2f = _sc_expand(
        u, v, usr_table, rel_table, adj_ent, adj_rel)
    sa0, sa1 = _sc_aggregate(v, ent_table, s0, s1, e1f, e2f)
    ue = ue.reshape(B, DIM)
    s0r = s0.reshape(B, NN)
    sa0 = sa0.reshape(B, DIM)
    sa1v = sa1.reshape(B, NN * DIM)
    bb = b.reshape(1, DIM)
    out = _head(ue, s0r, sa0, sa1v, W, bb)
    return out.reshape(B)


# speedup vs baseline: 39.9958x; 1.2907x over previous
import functools

import jax
import jax.numpy as jnp
from jax import lax
from jax.experimental import pallas as pl
from jax.experimental.pallas import tpu as pltpu
from jax.experimental.pallas import tpu_sc as plsc

NW = 32
DIM = 16
NN = 16
NR = 32


_SC_PARAMS = pltpu.CompilerParams(
    needs_layout_passes=False, use_tc_tiling_on_sc=False)


def _sc_relayout(usr_t, ent_t, adje_t, adjr_t):
    n = usr_t.shape[1]
    ch = 400
    nch = n // ch
    kmax = (nch + NW - 1) // NW
    mesh = plsc.VectorSubcoreMesh(core_axis_name="c", subcore_axis_name="s")

    @functools.partial(
        pl.kernel,
        out_type=[
            jax.ShapeDtypeStruct((n, DIM), jnp.float32),
            jax.ShapeDtypeStruct((n, DIM), jnp.float32),
            jax.ShapeDtypeStruct((n, DIM), jnp.int32),
            jax.ShapeDtypeStruct((n, DIM), jnp.int32),
        ],
        mesh=mesh,
        compiler_params=_SC_PARAMS,
        scratch_types=[
            pltpu.VMEM((2, DIM, ch), jnp.float32),
            pltpu.VMEM((2, DIM, ch), jnp.int32),
            pltpu.VMEM((3, ch, DIM), jnp.float32),
            pltpu.VMEM((3, ch, DIM), jnp.int32),
            pltpu.SemaphoreType.DMA,
            pltpu.SemaphoreType.DMA,
        ],
    )
    def body(usr_hbm, ent_hbm, adje_hbm, adjr_hbm,
             usr_o, ent_o, adje_o, adjr_o,
             binf, bini, boutf, bouti, semI, semO):
        wid = lax.axis_index("s") * 2 + lax.axis_index("c")
        dim_iota = lax.iota(jnp.int32, DIM)

        zero16 = jnp.zeros((DIM,), jnp.int32)

        def one_table(src, dst, bin_, bout):
            @pl.when(wid < nch)
            def _():
                pltpu.async_copy(
                    src.at[:, pl.ds(wid * ch, ch)], bin_.at[0], semI)

            def chunk_body(k, carry):
                c = k * NW + wid
                km = k % 2

                @pl.when(c < nch)
                def _():
                    pltpu.make_async_copy(
                        src.at[:, pl.ds(c * ch, ch)], bin_.at[km],
                        semI).wait()

                    @pl.when(c + NW < nch)
                    def _():
                        pltpu.async_copy(
                            src.at[:, pl.ds((c + NW) * ch, ch)],
                            bin_.at[(k + 1) % 2], semI)

                    kms = jnp.full((DIM,), km, jnp.int32)

                    @plsc.parallel_loop(0, ch, unroll=4)
                    def t_body(i):
                        bout[0, i] = plsc.load_gather(
                            bin_,
                            [kms, dim_iota,
                             jnp.full((DIM,), i, jnp.int32)])
                    pltpu.sync_copy(bout.at[0], dst.at[pl.ds(c * ch, ch)])
                return carry
            lax.fori_loop(0, kmax, chunk_body, 0)

        one_table(usr_hbm, usr_o, binf, boutf)
        one_table(ent_hbm, ent_o, binf, boutf)
        one_table(adje_hbm, adje_o, bini, bouti)
        one_table(adjr_hbm, adjr_o, bini, bouti)

    return body(usr_t, ent_t, adje_t, adjr_t)


def _sc_phase(u, v, usr_table, ent_table, rel_table, adj_ent, adj_rel):
    B = u.shape[0]
    nb = B // NW
    nc1 = nb * NN // 128
    ipc = 128 // NN
    mesh = plsc.VectorSubcoreMesh(core_axis_name="c", subcore_axis_name="s")

    @functools.partial(
        pl.kernel,
        out_type=[
            jax.ShapeDtypeStruct((NW, nb, DIM), jnp.float32),
            jax.ShapeDtypeStruct((NW, nb, NN), jnp.float32),
            jax.ShapeDtypeStruct((NW, nb, DIM), jnp.float32),
            jax.ShapeDtypeStruct((NW, nb, NN * DIM), jnp.float32),
        ],
        mesh=mesh,
        compiler_params=pltpu.CompilerParams(
            needs_layout_passes=False, use_tc_tiling_on_sc=False),
        scratch_types=[
            pltpu.VMEM((nb,), jnp.int32),
            pltpu.VMEM((nb,), jnp.int32),
            pltpu.VMEM((NR, DIM), jnp.float32),
            pltpu.VMEM((DIM, NR), jnp.float32),
            pltpu.VMEM((nb, NR), jnp.float32),
            pltpu.VMEM((nb, DIM), jnp.float32),
            pltpu.VMEM((nb, DIM), jnp.float32),
            pltpu.VMEM((nb, NN), jnp.int32),
            pltpu.VMEM((nb, NN), jnp.int32),
            pltpu.VMEM((nc1, 128), jnp.int32),
            pltpu.VMEM((nb, NN), jnp.float32),
            pltpu.VMEM((nb, DIM), jnp.float32),
            pltpu.VMEM((128, DIM), jnp.float32),
            pltpu.VMEM((128, NN), jnp.int32),
            pltpu.VMEM((128, NN), jnp.int32),
            pltpu.VMEM((NN, 128), jnp.int32),
            pltpu.VMEM((ipc, NN * DIM), jnp.float32),
            pltpu.VMEM((NN, 128, DIM), jnp.float32),
            pltpu.SemaphoreType.DMA,
        ],
    )
    def body(u_hbm, v_hbm, usr_hbm, ent_hbm, rel_hbm, adj_ent_hbm, adj_rel_hbm,
             ue_out, s0_out, sa0_out, sa1_out,
             u_v, v_v, rel_v, relT_v, srow_v, ue_v, ev0_v, r0_v, e1_v, e1f_v,
             s0_v, sa0_v, ev1c_v, r1c_v, e2c_v, e2f_c, sac_v, stage_v, sem):
        wid = lax.axis_index("s") * 2 + lax.axis_index("c")
        base = wid * nb

        pltpu.sync_copy(u_hbm.at[pl.ds(base, nb)], u_v)
        pltpu.sync_copy(v_hbm.at[pl.ds(base, nb)], v_v)
        pltpu.sync_copy(rel_hbm, rel_v)

        d2 = pltpu.async_copy(usr_hbm.at[u_v], ue_v, sem)
        d3 = pltpu.async_copy(ent_hbm.at[v_v], ev0_v, sem)
        d4 = pltpu.async_copy(adj_rel_hbm.at[v_v], r0_v, sem)
        d5 = pltpu.async_copy(adj_ent_hbm.at[v_v], e1_v, sem)

        dim_iota = lax.iota(jnp.int32, DIM)
        for r in range(NR):
            plsc.store_scatter(
                relT_v, [dim_iota, jnp.full((DIM,), r, dtype=jnp.int32)],
                rel_v[r])

        d2.wait(); d3.wait(); d4.wait(); d5.wait()
        pltpu.sync_copy(ue_v, ue_out.at[wid])

        def _tree_dot(uev, half):
            parts = []
            for q in range(4):
                t = uev[4 * q] * relT_v[4 * q, pl.ds(half * NN, NN)]
                for dd in range(4 * q + 1, 4 * q + 4):
                    t = t + uev[dd] * relT_v[dd, pl.ds(half * NN, NN)]
                parts.append(t)
            return (parts[0] + parts[1]) + (parts[2] + parts[3])

        @plsc.parallel_loop(0, nb, unroll=4)
        def srow_body(i):
            uev = ue_v[i]
            srow_v[i, pl.ds(0, NN)] = _tree_dot(uev, 0)
            srow_v[i, pl.ds(NN, NN)] = _tree_dot(uev, 1)

        @plsc.parallel_loop(0, nb, unroll=4)
        def s0_body(i):
            isplat = jnp.full((NN,), i, dtype=jnp.int32)
            s0_v[i] = plsc.load_gather(srow_v, [isplat, r0_v[i]])
        pltpu.sync_copy(s0_v, s0_out.at[wid])

        def e1f_body(j, _):
            for k in range(128 // NN):
                e1f_v[j, pl.ds(k * NN, NN)] = e1_v[j * (128 // NN) + k]
            return _
        lax.fori_loop(0, nc1, e1f_body, 0)

        def expz(svec):
            e = jnp.exp(svec - jnp.max(svec))
            return e, jnp.sum(e)

        def wsum(wv, rows):
            parts = []
            for q in range(4):
                t = wv[4 * q] * rows[4 * q]
                for n in range(4 * q + 1, 4 * q + 4):
                    t = t + wv[n] * rows[n]
                parts.append(t)
            return (parts[0] + parts[1]) + (parts[2] + parts[3])

        def lvl1_body(c, _):
            idx = e1f_v.at[c]
            g1 = pltpu.async_copy(ent_hbm.at[idx], ev1c_v, sem)
            g2 = pltpu.async_copy(adj_rel_hbm.at[idx], r1c_v, sem)
            g3 = pltpu.async_copy(adj_ent_hbm.at[idx], e2c_v, sem)
            g1.wait(); g2.wait(); g3.wait()

            @plsc.parallel_loop(0, ipc, unroll=ipc)
            def agg0_body(k):
                i = c * ipc + k
                ev, z = expz(s0_v[i])
                rows = [ev1c_v[k * NN + n] for n in range(NN)]
                sa0_v[i] = ev0_v[i] + wsum(ev, rows) / z

            @plsc.parallel_loop(0, NN, unroll=4)
            def e2f_body(jj):
                for k in range(128 // NN):
                    e2f_c[jj, pl.ds(k * NN, NN)] = e2c_v[jj * (128 // NN) + k]

            for jf in range(NN):
                pltpu.async_copy(
                    ent_hbm.at[e2f_c.at[jf]], stage_v.at[jf], sem)

            def lvl2_body(j, _2):
                pltpu.make_async_copy(
                    ent_hbm.at[e2f_c.at[j]], stage_v.at[j], sem).wait()

                @plsc.parallel_loop(0, ipc, unroll=ipc)
                def grp_body(k2):
                    gg = j * ipc + k2
                    isplat = jnp.full(
                        (NN,), c * ipc + gg // NN, dtype=jnp.int32)
                    svec = plsc.load_gather(srow_v, [isplat, r1c_v[gg]])
                    ev, z = expz(svec)
                    rows = [stage_v[j, k2 * NN + n] for n in range(NN)]
                    acc = ev1c_v[gg] + wsum(ev, rows) / z
                    sac_v[gg // NN, pl.ds((gg % NN) * DIM, DIM)] = acc
                return _2
            lax.fori_loop(0, NN, lvl2_body, 0)
            pltpu.sync_copy(sac_v, sa1_out.at[wid, pl.ds(c * ipc, ipc)])
            return _
        lax.fori_loop(0, nc1, lvl1_body, 0)
        pltpu.sync_copy(sa0_v, sa0_out.at[wid])

    return body(u, v, usr_table, ent_table, rel_table, adj_ent, adj_rel)


def _head_body(ue_ref, s0_ref, sa0_ref, sa1_ref, w_ref, b_ref, out_ref):
    W = w_ref[...]
    bvec = b_ref[...]
    f32 = jnp.float32
    E = (lax.broadcasted_iota(jnp.int32, (NN, NN * DIM), 1) // DIM ==
         lax.broadcasted_iota(jnp.int32, (NN, NN * DIM), 0)).astype(f32)
    S = (lax.broadcasted_iota(jnp.int32, (NN * DIM, DIM), 0) % DIM ==
         lax.broadcasted_iota(jnp.int32, (NN * DIM, DIM), 1)).astype(f32)
    P = (lax.broadcasted_iota(jnp.int32, (DIM, NN * DIM), 1) % DIM ==
         lax.broadcasted_iota(jnp.int32, (DIM, NN * DIM), 0)).astype(f32)
    blockmask = (
        lax.broadcasted_iota(jnp.int32, (NN * DIM, NN * DIM), 0) // DIM ==
        lax.broadcasted_iota(jnp.int32, (NN * DIM, NN * DIM), 1) // DIM
    ).astype(f32)
    BD = jnp.dot(S, jnp.dot(W, P, preferred_element_type=f32),
                 preferred_element_type=f32) * blockmask
    btile = jnp.dot(bvec, P, preferred_element_type=f32)

    sa1 = sa1_ref[...]
    h1v = jnp.maximum(
        jnp.dot(sa1, BD, preferred_element_type=f32) + btile, 0.0)
    h0 = jnp.maximum(
        jnp.dot(sa0_ref[...], W, preferred_element_type=f32) + bvec, 0.0)

    s0 = s0_ref[...]
    m = jnp.max(s0, axis=-1, keepdims=True)
    e = jnp.exp(s0 - m)
    w0 = e / jnp.sum(e, axis=-1, keepdims=True)
    t2 = jnp.dot(w0, E, preferred_element_type=f32) * h1v
    agg = jnp.dot(t2, S, preferred_element_type=f32)
    item = jnp.tanh(jnp.dot(h0 + agg, W, preferred_element_type=f32) + bvec)
    logit = jnp.sum(ue_ref[...] * item, axis=-1, keepdims=True)
    out_ref[...] = jax.nn.sigmoid(logit)


def _head(ue, s0, sa0, sa1v, W, bb):
    B = s0.shape[0]
    nt = 4
    tb = B // nt
    return pl.pallas_call(
        _head_body,
        grid=(nt,),
        in_specs=[
            pl.BlockSpec((tb, DIM), lambda i: (i, 0)),
            pl.BlockSpec((tb, NN), lambda i: (i, 0)),
            pl.BlockSpec((tb, DIM), lambda i: (i, 0)),
            pl.BlockSpec((tb, NN * DIM), lambda i: (i, 0)),
            pl.BlockSpec((DIM, DIM), lambda i: (0, 0)),
            pl.BlockSpec((1, DIM), lambda i: (0, 0)),
        ],
        out_specs=pl.BlockSpec((tb, 1), lambda i: (i, 0)),
        out_shape=jax.ShapeDtypeStruct((B, 1), jnp.float32),
    )(ue, s0, sa0, sa1v, W, bb)


def kernel(u, v, usr_table, ent_table, rel_table, adj_ent, adj_rel, W, b):
    B = u.shape[0]
    usr_rm, ent_rm, adje_rm, adjr_rm = _sc_relayout(
        jnp.swapaxes(usr_table, 0, 1), jnp.swapaxes(ent_table, 0, 1),
        jnp.swapaxes(adj_ent, 0, 1), jnp.swapaxes(adj_rel, 0, 1))
    ue, s0, sa0, sa1 = _sc_phase(
        u, v, usr_rm, ent_rm, rel_table, adje_rm, adjr_rm)
    ue = ue.reshape(B, DIM)
    s0 = s0.reshape(B, NN)
    sa0 = sa0.reshape(B, DIM)
    sa1v = sa1.reshape(B, NN * DIM)
    bb = b.reshape(1, DIM)
    out = _head(ue, s0, sa0, sa1v, W, bb)
    return out.reshape(B)


# speedup vs baseline: 42.6705x; 1.0669x over previous
import functools

import jax
import jax.numpy as jnp
from jax import lax
from jax.experimental import pallas as pl
from jax.experimental.pallas import tpu as pltpu
from jax.experimental.pallas import tpu_sc as plsc

NW = 32
DIM = 16
NN = 16
NR = 32


_SC_PARAMS = pltpu.CompilerParams(
    needs_layout_passes=False, use_tc_tiling_on_sc=False)


def _sc_relayout(usr_t, ent_t, adje_t, adjr_t):
    n = usr_t.shape[1]
    ch = 400
    nch = n // ch
    kmax = (nch + NW - 1) // NW
    mesh = plsc.VectorSubcoreMesh(core_axis_name="c", subcore_axis_name="s")

    @functools.partial(
        pl.kernel,
        out_type=[
            jax.ShapeDtypeStruct((n, DIM), jnp.float32),
            jax.ShapeDtypeStruct((n, DIM), jnp.float32),
            jax.ShapeDtypeStruct((n, DIM), jnp.int32),
            jax.ShapeDtypeStruct((n, DIM), jnp.int32),
        ],
        mesh=mesh,
        compiler_params=_SC_PARAMS,
        scratch_types=[
            pltpu.VMEM((2, DIM, ch), jnp.float32),
            pltpu.VMEM((2, DIM, ch), jnp.int32),
            pltpu.VMEM((3, ch, DIM), jnp.float32),
            pltpu.VMEM((3, ch, DIM), jnp.int32),
            pltpu.SemaphoreType.DMA,
            pltpu.SemaphoreType.DMA,
        ],
    )
    def body(usr_hbm, ent_hbm, adje_hbm, adjr_hbm,
             usr_o, ent_o, adje_o, adjr_o,
             binf, bini, boutf, bouti, semI, semO):
        wid = lax.axis_index("s") * 2 + lax.axis_index("c")
        dim_iota = lax.iota(jnp.int32, DIM)

        zero16 = jnp.zeros((DIM,), jnp.int32)

        def one_table(src, dst, bin_, bout):
            @pl.when(wid < nch)
            def _():
                pltpu.async_copy(
                    src.at[:, pl.ds(wid * ch, ch)], bin_.at[0], semI)

            def chunk_body(k, carry):
                c = k * NW + wid
                km = k % 2

                @pl.when(c < nch)
                def _():
                    pltpu.make_async_copy(
                        src.at[:, pl.ds(c * ch, ch)], bin_.at[km],
                        semI).wait()

                    @pl.when(c + NW < nch)
                    def _():
                        pltpu.async_copy(
                            src.at[:, pl.ds((c + NW) * ch, ch)],
                            bin_.at[(k + 1) % 2], semI)

                    kms = jnp.full((DIM,), km, jnp.int32)

                    @plsc.parallel_loop(0, ch, unroll=4)
                    def t_body(i):
                        bout[0, i] = plsc.load_gather(
                            bin_,
                            [kms, dim_iota,
                             jnp.full((DIM,), i, jnp.int32)])
                    pltpu.sync_copy(bout.at[0], dst.at[pl.ds(c * ch, ch)])
                return carry
            lax.fori_loop(0, kmax, chunk_body, 0)

        one_table(usr_hbm, usr_o, binf, boutf)
        one_table(ent_hbm, ent_o, binf, boutf)
        one_table(adje_hbm, adje_o, bini, bouti)
        one_table(adjr_hbm, adjr_o, bini, bouti)

    return body(usr_t, ent_t, adje_t, adjr_t)


def _sc_phase(u, v, usr_table, ent_table, rel_table, adj_ent, adj_rel):
    B = u.shape[0]
    nb = B // NW
    nc1 = nb * NN // 128
    ipc = 128 // NN
    mesh = plsc.VectorSubcoreMesh(core_axis_name="c", subcore_axis_name="s")

    @functools.partial(
        pl.kernel,
        out_type=[
            jax.ShapeDtypeStruct((NW, nb, DIM), jnp.float32),
            jax.ShapeDtypeStruct((NW, nb, NN), jnp.float32),
            jax.ShapeDtypeStruct((NW, nb, DIM), jnp.float32),
            jax.ShapeDtypeStruct((NW, nb, NN * DIM), jnp.float32),
        ],
        mesh=mesh,
        compiler_params=pltpu.CompilerParams(
            needs_layout_passes=False, use_tc_tiling_on_sc=False),
        scratch_types=[
            pltpu.VMEM((nb,), jnp.int32),
            pltpu.VMEM((nb,), jnp.int32),
            pltpu.VMEM((NR, DIM), jnp.float32),
            pltpu.VMEM((DIM, NR), jnp.float32),
            pltpu.VMEM((nb, NR), jnp.float32),
            pltpu.VMEM((nb, DIM), jnp.float32),
            pltpu.VMEM((nb, DIM), jnp.float32),
            pltpu.VMEM((nb, NN), jnp.int32),
            pltpu.VMEM((nb, NN), jnp.int32),
            pltpu.VMEM((nc1, 128), jnp.int32),
            pltpu.VMEM((nb, NN), jnp.float32),
            pltpu.VMEM((nb, DIM), jnp.float32),
            pltpu.VMEM((2, 128, DIM), jnp.float32),
            pltpu.VMEM((2, 128, NN), jnp.int32),
            pltpu.VMEM((2, 128, NN), jnp.int32),
            pltpu.VMEM((NN, 128), jnp.int32),
            pltpu.VMEM((ipc, NN * DIM), jnp.float32),
            pltpu.VMEM((NN, 128, DIM), jnp.float32),
            pltpu.SemaphoreType.DMA,
            pltpu.SemaphoreType.DMA,
        ],
    )
    def body(u_hbm, v_hbm, usr_hbm, ent_hbm, rel_hbm, adj_ent_hbm, adj_rel_hbm,
             ue_out, s0_out, sa0_out, sa1_out,
             u_v, v_v, rel_v, relT_v, srow_v, ue_v, ev0_v, r0_v, e1_v, e1f_v,
             s0_v, sa0_v, ev1c_v, r1c_v, e2c_v, e2f_c, sac_v, stage_v, sem,
             sem2):
        wid = lax.axis_index("s") * 2 + lax.axis_index("c")
        base = wid * nb

        pltpu.sync_copy(u_hbm.at[pl.ds(base, nb)], u_v)
        pltpu.sync_copy(v_hbm.at[pl.ds(base, nb)], v_v)
        pltpu.sync_copy(rel_hbm, rel_v)

        d2 = pltpu.async_copy(usr_hbm.at[u_v], ue_v, sem)
        d3 = pltpu.async_copy(ent_hbm.at[v_v], ev0_v, sem)
        d4 = pltpu.async_copy(adj_rel_hbm.at[v_v], r0_v, sem)
        d5 = pltpu.async_copy(adj_ent_hbm.at[v_v], e1_v, sem)

        dim_iota = lax.iota(jnp.int32, DIM)
        for r in range(NR):
            plsc.store_scatter(
                relT_v, [dim_iota, jnp.full((DIM,), r, dtype=jnp.int32)],
                rel_v[r])

        d2.wait(); d3.wait(); d4.wait(); d5.wait()
        pltpu.sync_copy(ue_v, ue_out.at[wid])

        def _tree_dot(uev, half):
            parts = []
            for q in range(4):
                t = uev[4 * q] * relT_v[4 * q, pl.ds(half * NN, NN)]
                for dd in range(4 * q + 1, 4 * q + 4):
                    t = t + uev[dd] * relT_v[dd, pl.ds(half * NN, NN)]
                parts.append(t)
            return (parts[0] + parts[1]) + (parts[2] + parts[3])

        @plsc.parallel_loop(0, nb, unroll=4)
        def srow_body(i):
            uev = ue_v[i]
            srow_v[i, pl.ds(0, NN)] = _tree_dot(uev, 0)
            srow_v[i, pl.ds(NN, NN)] = _tree_dot(uev, 1)

        @plsc.parallel_loop(0, nb, unroll=4)
        def s0_body(i):
            isplat = jnp.full((NN,), i, dtype=jnp.int32)
            s0_v[i] = plsc.load_gather(srow_v, [isplat, r0_v[i]])
        pltpu.sync_copy(s0_v, s0_out.at[wid])

        def e1f_body(j, _):
            for k in range(128 // NN):
                e1f_v[j, pl.ds(k * NN, NN)] = e1_v[j * (128 // NN) + k]
            return _
        lax.fori_loop(0, nc1, e1f_body, 0)

        def expz(svec):
            e = jnp.exp(svec - jnp.max(svec))
            return e, jnp.sum(e)

        def wsum(wv, rows):
            parts = []
            for q in range(4):
                t = wv[4 * q] * rows[4 * q]
                for n in range(4 * q + 1, 4 * q + 4):
                    t = t + wv[n] * rows[n]
                parts.append(t)
            return (parts[0] + parts[1]) + (parts[2] + parts[3])

        pltpu.async_copy(ent_hbm.at[e1f_v.at[0]], ev1c_v.at[0], sem)
        pltpu.async_copy(adj_rel_hbm.at[e1f_v.at[0]], r1c_v.at[0], sem)
        pltpu.async_copy(adj_ent_hbm.at[e1f_v.at[0]], e2c_v.at[0], sem)

        def lvl1_body(c, _):
            cm = c % 2
            idx = e1f_v.at[c]
            pltpu.make_async_copy(ent_hbm.at[idx], ev1c_v.at[cm], sem).wait()
            pltpu.make_async_copy(
                adj_rel_hbm.at[idx], r1c_v.at[cm], sem).wait()
            pltpu.make_async_copy(
                adj_ent_hbm.at[idx], e2c_v.at[cm], sem).wait()

            @pl.when(c < nc1 - 1)
            def _fire_next():
                cn = (c + 1) % 2
                nidx = e1f_v.at[c + 1]
                pltpu.async_copy(ent_hbm.at[nidx], ev1c_v.at[cn], sem)
                pltpu.async_copy(adj_rel_hbm.at[nidx], r1c_v.at[cn], sem)
                pltpu.async_copy(adj_ent_hbm.at[nidx], e2c_v.at[cn], sem)

            @plsc.parallel_loop(0, ipc, unroll=ipc)
            def agg0_body(k):
                i = c * ipc + k
                ev, z = expz(s0_v[i])
                rows = [ev1c_v[cm, k * NN + n] for n in range(NN)]
                sa0_v[i] = ev0_v[i] + wsum(ev, rows) / z

            @plsc.parallel_loop(0, NN, unroll=4)
            def e2f_body(jj):
                for k in range(128 // NN):
                    e2f_c[jj, pl.ds(k * NN, NN)] = (
                        e2c_v[cm, jj * (128 // NN) + k])

            for jf in range(NN):
                pltpu.async_copy(
                    ent_hbm.at[e2f_c.at[jf]], stage_v.at[jf], sem2)

            def lvl2_body(j, _2):
                pltpu.make_async_copy(
                    ent_hbm.at[e2f_c.at[j]], stage_v.at[j], sem2).wait()

                @plsc.parallel_loop(0, ipc, unroll=ipc)
                def grp_body(k2):
                    gg = j * ipc + k2
                    isplat = jnp.full(
                        (NN,), c * ipc + gg // NN, dtype=jnp.int32)
                    svec = plsc.load_gather(
                        srow_v, [isplat, r1c_v[cm, gg]])
                    ev, z = expz(svec)
                    rows = [stage_v[j, k2 * NN + n] for n in range(NN)]
                    acc = ev1c_v[cm, gg] + wsum(ev, rows) / z
                    sac_v[gg // NN, pl.ds((gg % NN) * DIM, DIM)] = acc
                return _2
            lax.fori_loop(0, NN, lvl2_body, 0)
            pltpu.sync_copy(sac_v, sa1_out.at[wid, pl.ds(c * ipc, ipc)])
            return _
        lax.fori_loop(0, nc1, lvl1_body, 0)
        pltpu.sync_copy(sa0_v, sa0_out.at[wid])

    return body(u, v, usr_table, ent_table, rel_table, adj_ent, adj_rel)


def _head_body(ue_ref, s0_ref, sa0_ref, sa1_ref, w_ref, b_ref, out_ref):
    W = w_ref[...]
    bvec = b_ref[...]
    f32 = jnp.float32
    E = (lax.broadcasted_iota(jnp.int32, (NN, NN * DIM), 1) // DIM ==
         lax.broadcasted_iota(jnp.int32, (NN, NN * DIM), 0)).astype(f32)
    S = (lax.broadcasted_iota(jnp.int32, (NN * DIM, DIM), 0) % DIM ==
         lax.broadcasted_iota(jnp.int32, (NN * DIM, DIM), 1)).astype(f32)
    P = (lax.broadcasted_iota(jnp.int32, (DIM, NN * DIM), 1) % DIM ==
         lax.broadcasted_iota(jnp.int32, (DIM, NN * DIM), 0)).astype(f32)
    blockmask = (
        lax.broadcasted_iota(jnp.int32, (NN * DIM, NN * DIM), 0) // DIM ==
        lax.broadcasted_iota(jnp.int32, (NN * DIM, NN * DIM), 1) // DIM
    ).astype(f32)
    BD = jnp.dot(S, jnp.dot(W, P, preferred_element_type=f32),
                 preferred_element_type=f32) * blockmask
    btile = jnp.dot(bvec, P, preferred_element_type=f32)

    sa1 = sa1_ref[...]
    h1v = jnp.maximum(
        jnp.dot(sa1, BD, preferred_element_type=f32) + btile, 0.0)
    h0 = jnp.maximum(
        jnp.dot(sa0_ref[...], W, preferred_element_type=f32) + bvec, 0.0)

    s0 = s0_ref[...]
    m = jnp.max(s0, axis=-1, keepdims=True)
    e = jnp.exp(s0 - m)
    w0 = e / jnp.sum(e, axis=-1, keepdims=True)
    t2 = jnp.dot(w0, E, preferred_element_type=f32) * h1v
    agg = jnp.dot(t2, S, preferred_element_type=f32)
    item = jnp.tanh(jnp.dot(h0 + agg, W, preferred_element_type=f32) + bvec)
    logit = jnp.sum(ue_ref[...] * item, axis=-1, keepdims=True)
    out_ref[...] = jax.nn.sigmoid(logit)


def _head(ue, s0, sa0, sa1v, W, bb):
    B = s0.shape[0]
    nt = 4
    tb = B // nt
    return pl.pallas_call(
        _head_body,
        grid=(nt,),
        in_specs=[
            pl.BlockSpec((tb, DIM), lambda i: (i, 0)),
            pl.BlockSpec((tb, NN), lambda i: (i, 0)),
            pl.BlockSpec((tb, DIM), lambda i: (i, 0)),
            pl.BlockSpec((tb, NN * DIM), lambda i: (i, 0)),
            pl.BlockSpec((DIM, DIM), lambda i: (0, 0)),
            pl.BlockSpec((1, DIM), lambda i: (0, 0)),
        ],
        out_specs=pl.BlockSpec((tb, 1), lambda i: (i, 0)),
        out_shape=jax.ShapeDtypeStruct((B, 1), jnp.float32),
    )(ue, s0, sa0, sa1v, W, bb)


def kernel(u, v, usr_table, ent_table, rel_table, adj_ent, adj_rel, W, b):
    B = u.shape[0]
    usr_rm, ent_rm, adje_rm, adjr_rm = _sc_relayout(
        jnp.swapaxes(usr_table, 0, 1), jnp.swapaxes(ent_table, 0, 1),
        jnp.swapaxes(adj_ent, 0, 1), jnp.swapaxes(adj_rel, 0, 1))
    ue, s0, sa0, sa1 = _sc_phase(
        u, v, usr_rm, ent_rm, rel_table, adje_rm, adjr_rm)
    ue = ue.reshape(B, DIM)
    s0 = s0.reshape(B, NN)
    sa0 = sa0.reshape(B, DIM)
    sa1v = sa1.reshape(B, NN * DIM)
    bb = b.reshape(1, DIM)
    out = _head(ue, s0, sa0, sa1v, W, bb)
    return out.reshape(B)
